# Initial kernel scaffold; baseline (speedup 1.0000x reference)
import jax
import jax.numpy as jnp
from jax.experimental import pallas as pl


def kernel(x, table):
    raise NotImplementedError("write your pallas kernel here")


# speedup vs baseline: 3.4143x; 3.4143x over previous
import functools

import jax
import jax.numpy as jnp
from jax import lax
from jax.experimental import pallas as pl
from jax.experimental.pallas import tpu as pltpu
from jax.experimental.pallas import tpu_sc as plsc

DIM = 64
NC, NS = 2, 16
NW = NC * NS
CHUNK = 128


@functools.cache
def _emb_call(n_total):
    n_per_w = n_total // NW
    n_chunks = n_per_w // CHUNK
    mesh = plsc.VectorSubcoreMesh(core_axis_name="c", subcore_axis_name="s")

    @functools.partial(
        pl.kernel,
        out_type=jax.ShapeDtypeStruct((n_total, DIM), jnp.float32),
        mesh=mesh,
        scratch_types=[
            pltpu.VMEM((n_chunks, CHUNK), jnp.int32),
            pltpu.VMEM((2, CHUNK, DIM), jnp.float32),
            pltpu.SemaphoreType.DMA,
        ],
        compiler_params=pltpu.CompilerParams(use_tc_tiling_on_sc=False),
    )
    def emb(idx_hbm, table_hbm, out_hbm, idx_v, rows_v, sem):
        wid = lax.axis_index("s") * NC + lax.axis_index("c")
        pltpu.sync_copy(idx_hbm.at[pl.ds(wid * n_chunks, n_chunks)], idx_v)
        base = wid * n_per_w

        def body(j, carry):
            pltpu.async_copy(table_hbm.at[idx_v.at[j]], rows_v.at[0], sem).wait()
            pltpu.sync_copy(rows_v.at[0],
                            out_hbm.at[pl.ds(base + j * CHUNK, CHUNK)])
            return carry

        lax.fori_loop(0, n_chunks, body, 0)

    return emb


def kernel(x, table):
    b, s = x.shape
    n_total = b * s
    idx = x.reshape(n_total // CHUNK, CHUNK).astype(jnp.int32)
    out = _emb_call(n_total)(idx, table.astype(jnp.float32))
    return out.reshape(b, s, DIM)


# speedup vs baseline: 3.5744x; 1.0469x over previous
import functools

import jax
import jax.numpy as jnp
from jax import lax
from jax.experimental import pallas as pl
from jax.experimental.pallas import tpu as pltpu
from jax.experimental.pallas import tpu_sc as plsc

DIM = 64
NC, NS = 2, 16
NW = NC * NS
CHUNK = 128
NBUF = 8
K = 4


@functools.cache
def _emb_call(n_total):
    n_per_w = n_total // NW
    n_chunks = n_per_w // CHUNK
    n_outer = n_chunks // NBUF
    assert n_chunks % NBUF == 0 and n_outer >= 2
    mesh = plsc.VectorSubcoreMesh(core_axis_name="c", subcore_axis_name="s")

    @functools.partial(
        pl.kernel,
        out_type=jax.ShapeDtypeStruct((n_total, DIM), jnp.float32),
        mesh=mesh,
        scratch_types=(
            [pltpu.VMEM((n_chunks, CHUNK), jnp.int32),
             pltpu.VMEM((NBUF, CHUNK, DIM), jnp.float32)]
            + [pltpu.SemaphoreType.DMA] * (2 * NBUF)
        ),
        compiler_params=pltpu.CompilerParams(use_tc_tiling_on_sc=False),
    )
    def emb(idx_hbm, table_hbm, out_hbm, idx_v, rows_v, *sems):
        sem_g, sem_s = sems[:NBUF], sems[NBUF:]
        wid = lax.axis_index("s") * NC + lax.axis_index("c")
        pltpu.sync_copy(idx_hbm.at[pl.ds(wid * n_chunks, n_chunks)], idx_v)
        base = wid * n_per_w

        def gather(j, b):
            pltpu.async_copy(table_hbm.at[idx_v.at[j]], rows_v.at[b],
                             sem_g[b])

        def gather_wait(j, b):
            pltpu.make_async_copy(table_hbm.at[idx_v.at[j]], rows_v.at[b],
                                  sem_g[b]).wait()

        def store(j, b):
            pltpu.async_copy(rows_v.at[b],
                             out_hbm.at[pl.ds(base + j * CHUNK, CHUNK)],
                             sem_s[b])

        def store_wait(j, b):
            pltpu.make_async_copy(rows_v.at[b],
                                  out_hbm.at[pl.ds(base + j * CHUNK, CHUNK)],
                                  sem_s[b]).wait()

        for b in range(K):
            gather(b, b)

        for b in range(NBUF):
            bb = (b + K) % NBUF
            if b >= NBUF - K:
                store_wait(b - (NBUF - K), bb)
            gather(b + K, bb)
            gather_wait(b, b)
            store(b, b)

        def outer(o, carry):
            j0 = o * NBUF
            for b in range(NBUF):
                j = j0 + b
                bb = (b + K) % NBUF
                store_wait(j - (NBUF - K), bb)
                gather(j + K, bb)
                gather_wait(j, b)
                store(j, b)
            return carry

        lax.fori_loop(1, n_outer - 1, outer, 0)

        j0 = (n_outer - 1) * NBUF
        for b in range(NBUF):
            j = j0 + b
            bb = (b + K) % NBUF
            if b < NBUF - K:
                store_wait(j - (NBUF - K), bb)
                gather(j + K, bb)
            gather_wait(j, b)
            store(j, b)

        for b in range(NBUF):
            store_wait(j0 + b, b)

    return emb


def kernel(x, table):
    b, s = x.shape
    n_total = b * s
    idx = x.reshape(n_total // CHUNK, CHUNK).astype(jnp.int32)
    out = _emb_call(n_total)(idx, table.astype(jnp.float32))
    return out.reshape(b, s, DIM)


# speedup vs baseline: 5.0307x; 1.4074x over previous
import functools

import jax
import jax.numpy as jnp
from jax import lax
from jax.experimental import pallas as pl
from jax.experimental.pallas import tpu as pltpu
from jax.experimental.pallas import tpu_sc as plsc

DIM = 64
NC, NS = 2, 16
NW = NC * NS
CHUNK = 128
NBUF = 8
K = 4


@functools.cache
def _emb_call(n_total):
    n_per_w = n_total // NW
    n_chunks = n_per_w // CHUNK
    n_outer = n_chunks // NBUF
    assert n_chunks % NBUF == 0 and n_outer >= 2
    mesh = plsc.VectorSubcoreMesh(core_axis_name="c", subcore_axis_name="s")

    @functools.partial(
        pl.kernel,
        out_type=jax.ShapeDtypeStruct((n_total, DIM), jnp.float32),
        mesh=mesh,
        scratch_types=(
            [pltpu.VMEM((n_chunks, CHUNK), jnp.int32),
             pltpu.VMEM((NBUF, CHUNK, DIM), jnp.float32),
             pltpu.VMEM_SHARED((1000, DIM), jnp.float32)]
            + [pltpu.SemaphoreType.DMA] * (2 * NBUF)
        ),
        compiler_params=pltpu.CompilerParams(use_tc_tiling_on_sc=False),
    )
    def emb(idx_hbm, table_hbm, out_hbm, idx_v, rows_v, table_sp, *sems):
        sem_g, sem_s = sems[:NBUF], sems[NBUF:]
        sid = lax.axis_index("s")
        wid = sid * NC + lax.axis_index("c")
        @pl.when(sid == 0)
        def _stage():
            pltpu.sync_copy(table_hbm, table_sp)

        pltpu.sync_copy(idx_hbm.at[pl.ds(wid * n_chunks, n_chunks)], idx_v)
        plsc.subcore_barrier()
        base = wid * n_per_w

        def gather(j, b):
            pltpu.async_copy(table_sp.at[idx_v.at[j]], rows_v.at[b],
                             sem_g[b])

        def gather_wait(j, b):
            pltpu.make_async_copy(table_sp.at[idx_v.at[j]], rows_v.at[b],
                                  sem_g[b]).wait()

        def store(j, b):
            pltpu.async_copy(rows_v.at[b],
                             out_hbm.at[pl.ds(base + j * CHUNK, CHUNK)],
                             sem_s[b])

        def store_wait(j, b):
            pltpu.make_async_copy(rows_v.at[b],
                                  out_hbm.at[pl.ds(base + j * CHUNK, CHUNK)],
                                  sem_s[b]).wait()

        for b in range(K):
            gather(b, b)

        for b in range(NBUF):
            bb = (b + K) % NBUF
            if b >= NBUF - K:
                store_wait(b - (NBUF - K), bb)
            gather(b + K, bb)
            gather_wait(b, b)
            store(b, b)

        def outer(o, carry):
            j0 = o * NBUF
            for b in range(NBUF):
                j = j0 + b
                bb = (b + K) % NBUF
                store_wait(j - (NBUF - K), bb)
                gather(j + K, bb)
                gather_wait(j, b)
                store(j, b)
            return carry

        lax.fori_loop(1, n_outer - 1, outer, 0)

        j0 = (n_outer - 1) * NBUF
        for b in range(NBUF):
            j = j0 + b
            bb = (b + K) % NBUF
            if b < NBUF - K:
                store_wait(j - (NBUF - K), bb)
                gather(j + K, bb)
            gather_wait(j, b)
            store(j, b)

        for b in range(NBUF):
            store_wait(j0 + b, b)

    return emb


def kernel(x, table):
    b, s = x.shape
    n_total = b * s
    idx = x.reshape(n_total // CHUNK, CHUNK).astype(jnp.int32)
    out = _emb_call(n_total)(idx, table.astype(jnp.float32))
    return out.reshape(b, s, DIM)
